# Initial kernel scaffold; baseline (speedup 1.0000x reference)
import jax
import jax.numpy as jnp
from jax.experimental import pallas as pl


def kernel(x, target_n_patches, W, b, gamma, beta):
    raise NotImplementedError("write your pallas kernel here")


# speedup vs baseline: 2700.4108x; 2700.4108x over previous
import functools

import jax
import jax.numpy as jnp
from jax import lax
from jax.experimental import pallas as pl
from jax.experimental.pallas import tpu as pltpu
from jax.experimental.pallas import tpu_sc as plsc

B = 8
L = 2048
PATCH = 16
T = 128
D_MODEL = 128
MIN_PATCH = 4
THRESHOLD_FACTOR = 1.5
EPS = 1e-5
NBLK = L // 16


def _gather(ref, idx):
    return plsc.load_gather(ref, [idx])


def _scatter(ref, idx, x, mask=None):
    plsc.store_scatter(ref, [idx], x, mask=mask)


def _wid():
    return lax.axis_index("s") * 2 + lax.axis_index("c")


_when = pl.when
_while = lax.while_loop
_fori = lax.fori_loop
_copy = pltpu.sync_copy


def _sc_body(x_hbm, seg_hbm, cnt_hbm, xr, dv, segv, bnd, cntv):
    wid = _wid()

    @_when(wid < B)
    def _():
        lanes = lax.iota(jnp.int32, 16)
        lanes_f = lanes.astype(jnp.float32)

        _copy(x_hbm.at[wid], xr.at[pl.ds(0, L)])

        def diff_blk(k, acc):
            a = xr[pl.ds(k * 16, 16)]
            bshift = _gather(xr, lanes + (k * 16 + 1))
            v = jnp.abs(bshift - a)
            p = lanes + k * 16
            ok = p < (L - 1)
            dv[pl.ds(k * 16, 16)] = jnp.where(ok, v, -1.0)
            return acc + jnp.where(ok, v, 0.0)

        acc = jnp.zeros((16,), jnp.float32)
        for k in range(NBLK):
            acc = diff_blk(k, acc)
        thr = jnp.sum(acc) * jnp.float32(THRESHOLD_FACTOR / (L - 1))

        for k in range(9):
            bnd[pl.ds(k * 16, 16)] = jnp.zeros((16,), jnp.int32)

        def cond(st):
            k, last, cnt = st
            return (k < NBLK) & (cnt <= T)

        def body(st):
            k, last, cnt = st
            d_blk = _gather(dv, lanes + k * 16)
            pos = lanes + (k * 16 + 1)
            m = (d_blk > thr) & (pos >= last + MIN_PATCH)
            has = jnp.any(m)
            cp = jnp.min(jnp.where(m, pos, jnp.int32(2 * L)))
            _scatter(
                bnd, jnp.full((16,), cnt, jnp.int32),
                jnp.full((16,), cp, jnp.int32),
                mask=(lanes == 0) & has)
            new_last = jnp.where(has, cp, last)
            new_cnt = cnt + has.astype(jnp.int32)
            new_k = jnp.where(has, k, k + 1)
            return (new_k, new_last, new_cnt)

        _, _, cnt = _while(
            cond, body, (jnp.int32(0), jnp.int32(0), jnp.int32(1)))

        _scatter(
            bnd, jnp.full((16,), cnt, jnp.int32),
            jnp.full((16,), jnp.int32(L), jnp.int32),
            mask=(lanes == 0) & (cnt <= T))

        def tok_blk(blk, _):
            s_v = _gather(bnd, lanes + blk * 16)
            e_v = _gather(bnd, lanes + blk * 16 + 1)
            seg_len = jnp.maximum(e_v - s_v, 1)
            sf = seg_len.astype(jnp.float32)
            scale = sf * (1.0 / PATCH)
            hi_cap = seg_len - 1
            hi_cap_f = hi_cap.astype(jnp.float32)
            valid = (lanes + blk * 16) < cnt
            for j in range(PATCH):
                src = (j + 0.5) * scale - 0.5
                src = jnp.minimum(jnp.maximum(src, 0.0), hi_cap_f)
                lo = src.astype(jnp.int32)
                hi = jnp.minimum(lo + 1, hi_cap)
                w = src - lo.astype(jnp.float32)
                gl = _gather(xr, s_v + lo)
                gh = _gather(xr, s_v + hi)
                res = gl * (1.0 - w) + gh * w
                res = jnp.where(valid, res, 0.0)
                _scatter(
                    segv, lanes + (blk * 16 + j * T), res)
            return 0

        _fori(0, T // 16, tok_blk, 0)

        cntv[...] = jnp.full((16,), cnt, jnp.int32)
        _copy(segv, seg_hbm.at[wid])
        _copy(cntv, cnt_hbm.at[wid])


_SC_STAGE_CACHE = {}


def _sc_stage_fn():
    if "k" not in _SC_STAGE_CACHE:
        _SC_STAGE_CACHE["k"] = pl.kernel(
            _sc_body,
            out_type=(
                jax.ShapeDtypeStruct((B, PATCH * T), jnp.float32),
                jax.ShapeDtypeStruct((B, 16), jnp.int32),
            ),
            mesh=plsc.VectorSubcoreMesh(
                core_axis_name="c", subcore_axis_name="s"),
            scratch_types=[
                pltpu.VMEM((L + 16,), jnp.float32),
                pltpu.VMEM((L,), jnp.float32),
                pltpu.VMEM((PATCH * T,), jnp.float32),
                pltpu.VMEM((144,), jnp.int32),
                pltpu.VMEM((16,), jnp.int32),
            ],
            compiler_params=pltpu.CompilerParams(needs_layout_passes=False),
        )
    return _SC_STAGE_CACHE["k"]


def _tc_body(seg_ref, cnt_ref, w_ref, b_ref, g_ref, beta_ref, tnp_ref, o_ref):
    seg = seg_ref[...]
    wmat = w_ref[...]
    pe = lax.dot_general(
        seg, wmat, (((1,), (0,)), ((), ())),
        preferred_element_type=jnp.float32)
    pe = pe + b_ref[...][None, None, :]
    cnt3 = lax.broadcast_in_dim(cnt_ref[:, 0:1], (B, T, D_MODEL), (0, 1))
    tok3 = lax.broadcasted_iota(jnp.int32, (B, T, D_MODEL), 1)
    valid3 = (tok3 < cnt3) & (tok3 < tnp_ref[0])
    pe = jnp.where(valid3, pe, 0.0)
    mean = jnp.mean(pe, axis=-1, keepdims=True)
    cen = pe - mean
    var = jnp.mean(cen * cen, axis=-1, keepdims=True)
    o_ref[...] = cen * lax.rsqrt(var + EPS) * g_ref[...] + beta_ref[...]


def kernel(x, target_n_patches, W, b, gamma, beta):
    seg, cnt = _sc_stage_fn()(x)
    seg3 = seg.reshape(B, PATCH, T)
    tnp = jnp.asarray(target_n_patches, jnp.int32).reshape(1)
    out = pl.pallas_call(
        _tc_body,
        out_shape=jax.ShapeDtypeStruct((B, T, D_MODEL), jnp.float32),
        in_specs=[
            pl.BlockSpec(memory_space=pltpu.VMEM),
            pl.BlockSpec(memory_space=pltpu.VMEM),
            pl.BlockSpec(memory_space=pltpu.VMEM),
            pl.BlockSpec(memory_space=pltpu.VMEM),
            pl.BlockSpec(memory_space=pltpu.VMEM),
            pl.BlockSpec(memory_space=pltpu.VMEM),
            pl.BlockSpec(memory_space=pltpu.SMEM),
        ],
    )(seg3, cnt, W, b, gamma, beta, tnp)
    return out
